# baseline (speedup 1.0000x reference)
import functools

import jax
import jax.numpy as jnp
from jax import lax
from jax.experimental import pallas as pl
from jax.experimental.pallas import tpu as pltpu
from jax.experimental.pallas import tpu_sc as plsc

B_ = 4096
C_ = 1000
L_ = 50
NC_ = 2
NS_ = 16
NW_ = NC_ * NS_
BPW_ = B_ // NW_
EPW_ = BPW_ * L_
SLAB_ = C_ * L_
CHUNK_ = 128
NCHUNK_ = EPW_ // CHUNK_
FILL_Q_ = 8


def _sc_onehot(tgt_hbm, out_hbm, zeros_v, tgt_v, idx_v, ones_v, fill_sem,
               scat_sem):
    wid = lax.axis_index("s") * NC_ + lax.axis_index("c")
    base_b = wid * BPW_
    base_e = wid * EPW_
    base_w = base_b * SLAB_

    def zbody(i, _):
        zeros_v[pl.ds(i * 16, 16)] = jnp.zeros((16,), jnp.int32)
        return 0
    lax.fori_loop(0, SLAB_ // 16, zbody, 0)
    for c in range(CHUNK_ // 16):
        ones_v[pl.ds(c * 16, 16)] = jnp.ones((16,), jnp.int32)

    pltpu.sync_copy(tgt_hbm.at[pl.ds(base_e, EPW_)], tgt_v)

    def ibody(j, _):
        for c in range(CHUNK_ // 16):
            k = j * CHUNK_ + c * 16 + lax.iota(jnp.int32, 16)
            bl = lax.div(k, L_)
            l = k - bl * L_
            t = tgt_v[pl.ds(j * CHUNK_ + c * 16, 16)]
            idx_v[j, pl.ds(c * 16, 16)] = (base_b + bl) * SLAB_ + t * L_ + l
        return 0
    lax.fori_loop(0, NCHUNK_, ibody, 0)

    def fire(i):
        pltpu.make_async_copy(
            zeros_v, out_hbm.at[pl.ds(base_w + i * SLAB_, SLAB_)],
            fill_sem).start()

    def drain(i):
        pltpu.make_async_copy(
            zeros_v, out_hbm.at[pl.ds(base_w + i * SLAB_, SLAB_)],
            fill_sem).wait()

    def fbody(i, _):
        fire(i)

        @pl.when(i >= FILL_Q_ - 1)
        def _():
            drain(i)
        return 0
    lax.fori_loop(0, BPW_, fbody, 0)

    def fdrain(i, _):
        drain(i)
        return 0
    lax.fori_loop(0, FILL_Q_ - 1, fdrain, 0)

    def sbody(j, _):
        pltpu.make_async_copy(ones_v, out_hbm.at[idx_v.at[j]],
                              scat_sem).start()
        return 0
    lax.fori_loop(0, NCHUNK_, sbody, 0)

    def sdrain(j, _):
        pltpu.make_async_copy(ones_v, out_hbm.at[idx_v.at[j]],
                              scat_sem).wait()
        return 0
    lax.fori_loop(0, NCHUNK_, sdrain, 0)


@jax.jit
def kernel(target):
    tgt_flat = jnp.reshape(target, (B_ * L_,))
    out_flat = pl.kernel(
        _sc_onehot,
        out_type=jax.ShapeDtypeStruct((B_ * C_ * L_,), jnp.int32),
        mesh=plsc.VectorSubcoreMesh(core_axis_name="c", subcore_axis_name="s"),
        scratch_types=[
            pltpu.VMEM((SLAB_,), jnp.int32),
            pltpu.VMEM((EPW_,), jnp.int32),
            pltpu.VMEM((NCHUNK_, CHUNK_), jnp.int32),
            pltpu.VMEM((CHUNK_,), jnp.int32),
            pltpu.SemaphoreType.DMA,
            pltpu.SemaphoreType.DMA,
        ],
    )(tgt_flat)
    return jnp.reshape(out_flat, (B_, C_, L_))
